# baseline (speedup 1.0000x reference)
import jax
import jax.numpy as jnp
from jax.experimental import pallas as pl
from jax.experimental.pallas import tpu as pltpu

_TOPK = 2
_BM = 1024
_NBUF = 4


def _top2(logits, idx_ref, gate_ref, j0):
    e, bm = logits.shape
    row = jax.lax.broadcasted_iota(jnp.int32, logits.shape, 0)
    l1 = jnp.max(logits, axis=0, keepdims=True)
    i1 = jnp.min(jnp.where(logits == l1, row, e), axis=0, keepdims=True)
    masked = jnp.where(row == i1, -jnp.inf, logits)
    l2 = jnp.max(masked, axis=0, keepdims=True)
    i2 = jnp.min(jnp.where(masked == l2, row, e), axis=0, keepdims=True)
    e2 = jnp.exp(l2 - l1)
    denom = 1.0 + e2
    idx_ref[:, j0:j0 + bm] = jnp.concatenate([i1, i2], axis=0)
    gate_ref[:, j0:j0 + bm] = jnp.concatenate(
        [1.0 / denom, e2 / denom], axis=0)


def _router_kernel(x_hbm, w_ref, idx_ref, gate_ref, xbuf, sem):
    m = x_hbm.shape[0]
    nsteps = m // _BM
    w = w_ref[...]

    def copy(i, slot):
        return pltpu.make_async_copy(
            x_hbm.at[pl.ds(i * _BM, _BM), :], xbuf.at[slot], sem.at[slot])

    for i in range(min(_NBUF, nsteps)):
        copy(i, i).start()
    for i in range(nsteps):
        slot = i % _NBUF
        copy(i, slot).wait()
        logits = jax.lax.dot_general(
            w, xbuf[slot], (((1,), (1,)), ((), ())),
            preferred_element_type=jnp.float32)
        _top2(logits, idx_ref, gate_ref, i * _BM)
        nxt = i + _NBUF
        if nxt < nsteps:
            copy(nxt, slot).start()


@jax.jit
def kernel(x, W):
    m, k = x.shape
    e = W.shape[0]
    idx_t, gates_t = pl.pallas_call(
        _router_kernel,
        in_specs=[
            pl.BlockSpec(memory_space=pl.ANY),
            pl.BlockSpec(memory_space=pltpu.VMEM),
        ],
        out_specs=[
            pl.BlockSpec(memory_space=pltpu.VMEM),
            pl.BlockSpec(memory_space=pltpu.VMEM),
        ],
        out_shape=[
            jax.ShapeDtypeStruct((_TOPK, m), jnp.int32),
            jax.ShapeDtypeStruct((_TOPK, m), jnp.float32),
        ],
        scratch_shapes=[
            pltpu.VMEM((_NBUF, _BM, k), jnp.float32),
            pltpu.SemaphoreType.DMA((_NBUF,)),
        ],
    )(x, W)
    return idx_t.T, gates_t.T


# speedup vs baseline: 1.0082x; 1.0082x over previous
import jax
import jax.numpy as jnp
from jax.experimental import pallas as pl
from jax.experimental.pallas import tpu as pltpu

_TOPK = 2
_BM = 512
_NBUF = 8


def _top2(logits, idx_ref, gate_ref, j0):
    e, bm = logits.shape
    row = jax.lax.broadcasted_iota(jnp.int32, logits.shape, 0)
    l1 = jnp.max(logits, axis=0, keepdims=True)
    i1 = jnp.min(jnp.where(logits == l1, row, e), axis=0, keepdims=True)
    masked = jnp.where(row == i1, -jnp.inf, logits)
    l2 = jnp.max(masked, axis=0, keepdims=True)
    i2 = jnp.min(jnp.where(masked == l2, row, e), axis=0, keepdims=True)
    e2 = jnp.exp(l2 - l1)
    denom = 1.0 + e2
    idx_ref[:, j0:j0 + bm] = jnp.concatenate([i1, i2], axis=0)
    gate_ref[:, j0:j0 + bm] = jnp.concatenate(
        [1.0 / denom, e2 / denom], axis=0)


def _router_kernel(x_hbm, w_ref, idx_ref, gate_ref, xbuf, sem):
    m = x_hbm.shape[0]
    nsteps = m // _BM
    w = w_ref[...]

    def copy(i, slot):
        return pltpu.make_async_copy(
            x_hbm.at[pl.ds(i * _BM, _BM), :], xbuf.at[slot], sem.at[slot])

    for i in range(min(_NBUF, nsteps)):
        copy(i, i).start()
    for i in range(nsteps):
        slot = i % _NBUF
        copy(i, slot).wait()
        logits = jax.lax.dot_general(
            w, xbuf[slot], (((1,), (1,)), ((), ())),
            preferred_element_type=jnp.float32)
        _top2(logits, idx_ref, gate_ref, i * _BM)
        nxt = i + _NBUF
        if nxt < nsteps:
            copy(nxt, slot).start()


@jax.jit
def kernel(x, W):
    m, k = x.shape
    e = W.shape[0]
    idx_t, gates_t = pl.pallas_call(
        _router_kernel,
        in_specs=[
            pl.BlockSpec(memory_space=pl.ANY),
            pl.BlockSpec(memory_space=pltpu.VMEM),
        ],
        out_specs=[
            pl.BlockSpec(memory_space=pltpu.VMEM),
            pl.BlockSpec(memory_space=pltpu.VMEM),
        ],
        out_shape=[
            jax.ShapeDtypeStruct((_TOPK, m), jnp.int32),
            jax.ShapeDtypeStruct((_TOPK, m), jnp.float32),
        ],
        scratch_shapes=[
            pltpu.VMEM((_NBUF, _BM, k), jnp.float32),
            pltpu.SemaphoreType.DMA((_NBUF,)),
        ],
    )(x, W)
    return idx_t.T, gates_t.T


# speedup vs baseline: 1.0471x; 1.0386x over previous
import jax
import jax.numpy as jnp
from jax.experimental import pallas as pl

_TOPK = 2
_BM = 1024


def _router_kernel(x_ref, w_ref, idx_ref, gate_ref):
    x = x_ref[...]
    w = w_ref[...]
    logits = jax.lax.dot_general(
        w, x, (((1,), (1,)), ((), ())),
        preferred_element_type=jnp.float32)
    e = logits.shape[0]
    row = jax.lax.broadcasted_iota(jnp.int32, logits.shape, 0)

    l1 = jnp.max(logits, axis=0, keepdims=True)
    i1 = jnp.min(jnp.where(logits == l1, row, e), axis=0, keepdims=True)
    masked = jnp.where(row == i1, -jnp.inf, logits)
    l2 = jnp.max(masked, axis=0, keepdims=True)
    i2 = jnp.min(jnp.where(masked == l2, row, e), axis=0, keepdims=True)

    e2 = jnp.exp(l2 - l1)
    denom = 1.0 + e2
    idx_ref[...] = jnp.concatenate([i1, i2], axis=0)
    gate_ref[...] = jnp.concatenate([1.0 / denom, e2 / denom], axis=0)


@jax.jit
def kernel(x, W):
    m, k = x.shape
    e = W.shape[0]
    grid = (m // _BM,)
    idx_t, gates_t = pl.pallas_call(
        _router_kernel,
        grid=grid,
        in_specs=[
            pl.BlockSpec((_BM, k), lambda i: (i, 0)),
            pl.BlockSpec((e, k), lambda i: (0, 0)),
        ],
        out_specs=[
            pl.BlockSpec((_TOPK, _BM), lambda i: (0, i)),
            pl.BlockSpec((_TOPK, _BM), lambda i: (0, i)),
        ],
        out_shape=[
            jax.ShapeDtypeStruct((_TOPK, m), jnp.int32),
            jax.ShapeDtypeStruct((_TOPK, m), jnp.float32),
        ],
    )(x, W)
    return idx_t.T, gates_t.T
